# baseline (speedup 1.0000x reference)
import functools

import jax
import jax.numpy as jnp
import numpy as np
from jax import lax
from jax.experimental import pallas as pl
from jax.experimental.pallas import tpu as pltpu
from jax.experimental.pallas import tpu_sc as plsc

N = 10000
E = 320000
D = 128
DE = 16
H = 128

NC = 2
NS = 16
NW = NC * NS
EPW = E // NW
CH = 40
NIT = EPW // CH
ZR = 624
ZR_LAST = N - (NS - 1) * ZR

_LOG2 = 0.6931471805599453


def _ssp(x):
    return jnp.maximum(x, 0.0) + jnp.log1p(jnp.exp(-jnp.abs(x))) - _LOG2


def _node_proj_body(nf_ref, w_ref, b_ref, out_ref):
    out_ref[...] = (
        jnp.dot(nf_ref[...], w_ref[...], preferred_element_type=jnp.float32)
        + b_ref[...]
    )


def _edge_mlp_body(eft_lo_ref, eft_hi_ref, w1_ref, b1_ref, w2_ref, b2_ref,
                   out_ref):
    def mlp(x):
        t = lax.dot_general(x, w1_ref[...], (((0,), (0,)), ((), ())),
                            preferred_element_type=jnp.float32)
        t = _ssp(t + b1_ref[...])
        t = jnp.dot(t, w2_ref[...], preferred_element_type=jnp.float32)
        return _ssp(t + b2_ref[...])

    lo = lax.bitcast_convert_type(
        mlp(eft_lo_ref[...]).astype(jnp.bfloat16), jnp.uint16).astype(jnp.int32)
    hi = lax.bitcast_convert_type(
        mlp(eft_hi_ref[...]).astype(jnp.bfloat16), jnp.uint16).astype(jnp.int32)
    out_ref[...] = lax.bitwise_or(lo, lax.shift_left(hi, 16))


def _out_proj_body(p0_ref, p1_ref, wpo_ref, bpo_ref, wout_ref, bout_ref, out_ref):
    agg = p0_ref[...] + p1_ref[...]
    h = _ssp(
        jnp.dot(agg, wpo_ref[...], preferred_element_type=jnp.float32)
        + bpo_ref[...]
    )
    out_ref[...] = (
        jnp.dot(h, wout_ref[...], preferred_element_type=jnp.float32)
        + bout_ref[...]
    )


RN = 2000
REH = 1280


def _node_proj(nf, w, b):
    return pl.pallas_call(
        _node_proj_body,
        grid=(N // RN,),
        in_specs=[
            pl.BlockSpec((RN, D), lambda i: (i, 0)),
            pl.BlockSpec((D, H), lambda i: (0, 0)),
            pl.BlockSpec((1, H), lambda i: (0, 0)),
        ],
        out_specs=pl.BlockSpec((RN, H), lambda i: (i, 0)),
        out_shape=jax.ShapeDtypeStruct((N, H), jnp.float32),
    )(nf, w, b)


NBH = E // (2 * REH)


def _edge_mlp(eft, w1, b1, w2, b2):
    return pl.pallas_call(
        _edge_mlp_body,
        grid=(NBH,),
        in_specs=[
            pl.BlockSpec((DE, REH), lambda i: (0, i)),
            pl.BlockSpec((DE, REH), lambda i: (0, i + NBH)),
            pl.BlockSpec((DE, H), lambda i: (0, 0)),
            pl.BlockSpec((1, H), lambda i: (0, 0)),
            pl.BlockSpec((H, H), lambda i: (0, 0)),
            pl.BlockSpec((1, H), lambda i: (0, 0)),
        ],
        out_specs=pl.BlockSpec((REH, H), lambda i: (i, 0)),
        out_shape=jax.ShapeDtypeStruct((E // 2, H), jnp.int32),
    )(eft, eft, w1, b1, w2, b2)


def _out_proj(p0, p1, wpo, bpo, wout, bout):
    return pl.pallas_call(
        _out_proj_body,
        grid=(N // RN,),
        in_specs=[
            pl.BlockSpec((RN, H), lambda i: (i, 0)),
            pl.BlockSpec((RN, H), lambda i: (i, 0)),
            pl.BlockSpec((H, D), lambda i: (0, 0)),
            pl.BlockSpec((1, D), lambda i: (0, 0)),
            pl.BlockSpec((D, D), lambda i: (0, 0)),
            pl.BlockSpec((1, D), lambda i: (0, 0)),
        ],
        out_specs=pl.BlockSpec((RN, D), lambda i: (i, 0)),
        out_shape=jax.ShapeDtypeStruct((N, D), jnp.float32),
    )(p0, p1, wpo, bpo, wout, bout)


def _sc_gather_mul_scatter(hv, he, src3, dst3, zinit):
    mesh = plsc.VectorSubcoreMesh(core_axis_name="c", subcore_axis_name="s")

    @functools.partial(
        pl.kernel,
        mesh=mesh,
        out_type=[
            jax.ShapeDtypeStruct((N, H), jnp.float32),
            jax.ShapeDtypeStruct((N, H), jnp.float32),
        ],
        scratch_types=[
            pltpu.VMEM((EPW,), jnp.int32),
            pltpu.VMEM((CH,), jnp.int32),
            pltpu.VMEM((CH,), jnp.int32),
            pltpu.VMEM((CH // 2 * H,), jnp.int32),
            pltpu.VMEM((CH // 2 * H,), jnp.int32),
            pltpu.VMEM((CH, H), jnp.float32),
            pltpu.VMEM((CH, H), jnp.float32),
            pltpu.VMEM((CH, H), jnp.float32),
            pltpu.VMEM_SHARED((N, H), jnp.float32),
            pltpu.SemaphoreType.DMA,
            pltpu.SemaphoreType.DMA,
            pltpu.SemaphoreType.DMA,
            pltpu.SemaphoreType.DMA,
            pltpu.SemaphoreType.DMA,
        ],
    )
    def k(hv_hbm, he_hbm, src_hbm, dst_hbm, z_hbm, out0, out1,
          src_i, d0, d1, he0, he1, hvr0, hvr1, pr, agg_sh,
          g0, g1, h0, h1, s0):
        c = lax.axis_index("c")
        s = lax.axis_index("s")
        wid = s * NC + c
        row0 = s * ZR
        ebase = wid * EPW

        def drain_f32(sem, buf):
            pltpu.make_async_copy(hv_hbm.at[pl.ds(0, CH)], buf, sem).wait()

        def drain_he(sem, buf):
            pltpu.make_async_copy(
                he_hbm.at[pl.ds(0, CH // 2 * H)], buf, sem).wait()

        @pl.when(s < NS - 1)
        def _():
            pltpu.sync_copy(z_hbm.at[pl.ds(row0, ZR)], agg_sh.at[pl.ds(row0, ZR)])

        @pl.when(s == NS - 1)
        def _():
            pltpu.sync_copy(z_hbm.at[pl.ds((NS - 1) * ZR, ZR_LAST)],
                            agg_sh.at[pl.ds((NS - 1) * ZR, ZR_LAST)])

        pltpu.sync_copy(src_hbm.at[pl.ds(ebase, EPW)], src_i)
        plsc.subcore_barrier()

        pltpu.async_copy(hv_hbm.at[src_i.at[pl.ds(0, CH)]], hvr0, g0)
        pltpu.async_copy(he_hbm.at[pl.ds(ebase // 2 * H, CH // 2 * H)], he0, h0)
        pltpu.async_copy(dst_hbm.at[pl.ds(ebase, CH)], d0, h0)

        def process(i, first, hvr, he_b, d_b, gsem, hsem, n_hvr, n_he, n_d, n_g, n_h):
            @pl.when(i + 1 < NIT)
            def _():
                pltpu.async_copy(
                    hv_hbm.at[src_i.at[pl.ds((i + 1) * CH, CH)]], n_hvr, n_g)
                pltpu.async_copy(
                    he_hbm.at[pl.ds((ebase + (i + 1) * CH) // 2 * H,
                                    CH // 2 * H)], n_he, n_h)
                pltpu.async_copy(dst_hbm.at[pl.ds(ebase + (i + 1) * CH, CH)],
                                 n_d, n_h)

            drain_f32(gsem, hvr)
            drain_he(hsem, he_b)
            pltpu.make_async_copy(dst_hbm.at[pl.ds(0, CH)], d_b, hsem).wait()

            @pl.when(jnp.logical_not(first))
            def _():
                drain_f32(s0, pr)

            def pair(k, cr):
                for j in range(H // 16):
                    w = he_b[pl.ds(k * H + 16 * j, 16)]
                    a = lax.bitcast_convert_type(
                        lax.shift_left(w, 16), jnp.float32)
                    b = lax.bitcast_convert_type(
                        lax.bitwise_and(w, jnp.int32(-65536)), jnp.float32)
                    sl = pl.ds(16 * j, 16)
                    pr[2 * k, sl] = hvr[2 * k, sl] * a
                    pr[2 * k + 1, sl] = hvr[2 * k + 1, sl] * b
                return cr

            lax.fori_loop(0, CH // 2, pair, 0)
            pltpu.async_copy(pr, agg_sh.at[d_b], s0, add=True)

        def outer(io, carry):
            i0 = io * 2
            process(i0, io == 0, hvr0, he0, d0, g0, h0, hvr1, he1, d1, g1, h1)
            process(i0 + 1, jnp.bool_(False), hvr1, he1, d1, g1, h1,
                    hvr0, he0, d0, g0, h0)
            return carry

        lax.fori_loop(0, NIT // 2, outer, 0)
        drain_f32(s0, pr)
        plsc.subcore_barrier()

        @pl.when((c == 0) & (s < NS - 1))
        def _():
            pltpu.sync_copy(agg_sh.at[pl.ds(row0, ZR)], out0.at[pl.ds(row0, ZR)])

        @pl.when((c == 0) & (s == NS - 1))
        def _():
            pltpu.sync_copy(agg_sh.at[pl.ds((NS - 1) * ZR, ZR_LAST)],
                            out0.at[pl.ds((NS - 1) * ZR, ZR_LAST)])

        @pl.when((c == 1) & (s < NS - 1))
        def _():
            pltpu.sync_copy(agg_sh.at[pl.ds(row0, ZR)], out1.at[pl.ds(row0, ZR)])

        @pl.when((c == 1) & (s == NS - 1))
        def _():
            pltpu.sync_copy(agg_sh.at[pl.ds((NS - 1) * ZR, ZR_LAST)],
                            out1.at[pl.ds((NS - 1) * ZR, ZR_LAST)])

    return k(hv, he, src3, dst3, zinit)


def kernel(node_feats, edge_feats, edge_index, Wpe1, bpe1, Wpe2, bpe2,
           Wpn, bpn, Wpo, bpo, Wout, bout):
    src_p = edge_index[0].reshape(2, E // 2).transpose(1, 0).reshape(E)
    dst_p = edge_index[1].reshape(2, E // 2).transpose(1, 0).reshape(E)
    hv = _node_proj(node_feats, Wpn, bpn.reshape(1, H))
    he = _edge_mlp(edge_feats.T, Wpe1, bpe1.reshape(1, H),
                   Wpe2, bpe2.reshape(1, H))
    he1d = he.reshape(E // 2 * H)
    zinit = jnp.zeros((N, H), jnp.float32)
    p0, p1 = _sc_gather_mul_scatter(hv, he1d, src_p, dst_p, zinit)
    return _out_proj(p0, p1, Wpo, bpo.reshape(1, H), Wout, bout.reshape(1, D))


# speedup vs baseline: 1.8632x; 1.8632x over previous
import functools

import jax
import jax.numpy as jnp
import numpy as np
from jax import lax
from jax.experimental import pallas as pl
from jax.experimental.pallas import tpu as pltpu
from jax.experimental.pallas import tpu_sc as plsc

N = 10000
E = 320000
D = 128
DE = 16
H = 128

NC = 2
NS = 16
NW = NC * NS
EPW = E // (2 * NW)
CH = 40
NIT = EPW // CH
ZR = 624
ZR_LAST = N - (NS - 1) * ZR

_LOG2 = 0.6931471805599453


def _ssp(x):
    return jnp.maximum(x, 0.0) + jnp.log1p(jnp.exp(-jnp.abs(x))) - _LOG2


def _node_proj_body(nf_ref, w_ref, b_ref, out_ref):
    out_ref[...] = (
        jnp.dot(nf_ref[...], w_ref[...], preferred_element_type=jnp.float32)
        + b_ref[...]
    )


def _edge_mlp_body(eft_ref, w1_ref, b1_ref, w2_ref, b2_ref, out_ref):
    t = lax.dot_general(eft_ref[...], w1_ref[...], (((0,), (0,)), ((), ())),
                        preferred_element_type=jnp.float32)
    t = _ssp(t + b1_ref[...])
    t = jnp.dot(t, w2_ref[...], preferred_element_type=jnp.float32)
    out_ref[...] = _ssp(t + b2_ref[...])


def _out_proj_body(p0_ref, p1_ref, wpo_ref, bpo_ref, wout_ref, bout_ref, out_ref):
    agg = p0_ref[...] + p1_ref[...]
    h = _ssp(
        jnp.dot(agg, wpo_ref[...], preferred_element_type=jnp.float32)
        + bpo_ref[...]
    )
    out_ref[...] = (
        jnp.dot(h, wout_ref[...], preferred_element_type=jnp.float32)
        + bout_ref[...]
    )


RN = 2000
REH = 1280


def _node_proj(nf, w, b):
    return pl.pallas_call(
        _node_proj_body,
        grid=(N // RN,),
        in_specs=[
            pl.BlockSpec((RN, D), lambda i: (i, 0)),
            pl.BlockSpec((D, H), lambda i: (0, 0)),
            pl.BlockSpec((1, H), lambda i: (0, 0)),
        ],
        out_specs=pl.BlockSpec((RN, H), lambda i: (i, 0)),
        out_shape=jax.ShapeDtypeStruct((N, H), jnp.float32),
    )(nf, w, b)


NBH = E // (2 * REH)


def _edge_mlp_half(eft, w1, b1, w2, b2, half):
    return pl.pallas_call(
        _edge_mlp_body,
        grid=(NBH,),
        in_specs=[
            pl.BlockSpec((DE, REH), lambda i: (0, i + half * NBH)),
            pl.BlockSpec((DE, H), lambda i: (0, 0)),
            pl.BlockSpec((1, H), lambda i: (0, 0)),
            pl.BlockSpec((H, H), lambda i: (0, 0)),
            pl.BlockSpec((1, H), lambda i: (0, 0)),
        ],
        out_specs=pl.BlockSpec((REH, H), lambda i: (i, 0)),
        out_shape=jax.ShapeDtypeStruct((E // 2, H), jnp.float32),
    )(eft, w1, b1, w2, b2)


def _out_proj(p0, p1, wpo, bpo, wout, bout):
    return pl.pallas_call(
        _out_proj_body,
        grid=(N // RN,),
        in_specs=[
            pl.BlockSpec((RN, H), lambda i: (i, 0)),
            pl.BlockSpec((RN, H), lambda i: (i, 0)),
            pl.BlockSpec((H, D), lambda i: (0, 0)),
            pl.BlockSpec((1, D), lambda i: (0, 0)),
            pl.BlockSpec((D, D), lambda i: (0, 0)),
            pl.BlockSpec((1, D), lambda i: (0, 0)),
        ],
        out_specs=pl.BlockSpec((RN, D), lambda i: (i, 0)),
        out_shape=jax.ShapeDtypeStruct((N, D), jnp.float32),
    )(p0, p1, wpo, bpo, wout, bout)


def _sc_gather_mul_scatter(hv, he, src3, dst3, zinit, half):
    mesh = plsc.VectorSubcoreMesh(core_axis_name="c", subcore_axis_name="s")

    @functools.partial(
        pl.kernel,
        mesh=mesh,
        out_type=[
            jax.ShapeDtypeStruct((N, H), jnp.float32),
            jax.ShapeDtypeStruct((N, H), jnp.float32),
        ],
        scratch_types=[
            pltpu.VMEM((EPW,), jnp.int32),
            pltpu.VMEM((CH,), jnp.int32),
            pltpu.VMEM((CH,), jnp.int32),
            pltpu.VMEM((CH, H), jnp.float32),
            pltpu.VMEM((CH, H), jnp.float32),
            pltpu.VMEM((CH, H), jnp.float32),
            pltpu.VMEM((CH, H), jnp.float32),
            pltpu.VMEM((CH, H), jnp.float32),
            pltpu.VMEM_SHARED((N, H), jnp.float32),
            pltpu.SemaphoreType.DMA,
            pltpu.SemaphoreType.DMA,
            pltpu.SemaphoreType.DMA,
            pltpu.SemaphoreType.DMA,
            pltpu.SemaphoreType.DMA,
        ],
    )
    def k(hv_hbm, he_hbm, src_hbm, dst_hbm, z_hbm, out0, out1,
          src_i, d0, d1, he0, he1, hvr0, hvr1, pr, agg_sh,
          g0, g1, h0, h1, s0):
        c = lax.axis_index("c")
        s = lax.axis_index("s")
        wid = s * NC + c
        row0 = s * ZR
        ebase = half * (E // 2) + wid * EPW
        hbase = wid * EPW

        def drain_f32(sem, buf):
            pltpu.make_async_copy(hv_hbm.at[pl.ds(0, CH)], buf, sem).wait()

        @pl.when(s < NS - 1)
        def _():
            pltpu.sync_copy(z_hbm.at[pl.ds(row0, ZR)], agg_sh.at[pl.ds(row0, ZR)])

        @pl.when(s == NS - 1)
        def _():
            pltpu.sync_copy(z_hbm.at[pl.ds((NS - 1) * ZR, ZR_LAST)],
                            agg_sh.at[pl.ds((NS - 1) * ZR, ZR_LAST)])

        pltpu.sync_copy(src_hbm.at[pl.ds(ebase, EPW)], src_i)
        plsc.subcore_barrier()

        pltpu.async_copy(hv_hbm.at[src_i.at[pl.ds(0, CH)]], hvr0, g0)
        pltpu.async_copy(he_hbm.at[pl.ds(hbase, CH)], he0, h0)
        pltpu.async_copy(dst_hbm.at[pl.ds(ebase, CH)], d0, h0)

        def process(i, first, hvr, he_b, d_b, gsem, hsem, n_hvr, n_he, n_d, n_g, n_h):
            @pl.when(i + 1 < NIT)
            def _():
                pltpu.async_copy(
                    hv_hbm.at[src_i.at[pl.ds((i + 1) * CH, CH)]], n_hvr, n_g)
                pltpu.async_copy(he_hbm.at[pl.ds(hbase + (i + 1) * CH, CH)],
                                 n_he, n_h)
                pltpu.async_copy(dst_hbm.at[pl.ds(ebase + (i + 1) * CH, CH)],
                                 n_d, n_h)

            drain_f32(gsem, hvr)
            drain_f32(hsem, he_b)
            pltpu.make_async_copy(dst_hbm.at[pl.ds(0, CH)], d_b, hsem).wait()

            @pl.when(jnp.logical_not(first))
            def _():
                drain_f32(s0, pr)

            def row(r, cr):
                for j in range(H // 16):
                    sl = pl.ds(j * 16, 16)
                    pr[r, sl] = hvr[r, sl] * he_b[r, sl]
                return cr

            lax.fori_loop(0, CH, row, 0)
            pltpu.async_copy(pr, agg_sh.at[d_b], s0, add=True)

        def outer(io, carry):
            i0 = io * 2
            process(i0, io == 0, hvr0, he0, d0, g0, h0, hvr1, he1, d1, g1, h1)
            process(i0 + 1, jnp.bool_(False), hvr1, he1, d1, g1, h1,
                    hvr0, he0, d0, g0, h0)
            return carry

        lax.fori_loop(0, NIT // 2, outer, 0)
        process(jnp.int32(NIT - 1), jnp.bool_(False), hvr0, he0, d0, g0, h0,
                hvr1, he1, d1, g1, h1)
        drain_f32(s0, pr)
        plsc.subcore_barrier()

        @pl.when((c == 0) & (s < NS - 1))
        def _():
            pltpu.sync_copy(agg_sh.at[pl.ds(row0, ZR)], out0.at[pl.ds(row0, ZR)])

        @pl.when((c == 0) & (s == NS - 1))
        def _():
            pltpu.sync_copy(agg_sh.at[pl.ds((NS - 1) * ZR, ZR_LAST)],
                            out0.at[pl.ds((NS - 1) * ZR, ZR_LAST)])

        @pl.when((c == 1) & (s < NS - 1))
        def _():
            pltpu.sync_copy(agg_sh.at[pl.ds(row0, ZR)], out1.at[pl.ds(row0, ZR)])

        @pl.when((c == 1) & (s == NS - 1))
        def _():
            pltpu.sync_copy(agg_sh.at[pl.ds((NS - 1) * ZR, ZR_LAST)],
                            out1.at[pl.ds((NS - 1) * ZR, ZR_LAST)])

    return k(hv, he, src3, dst3, zinit)


def _out_proj4_body(p0_ref, p1_ref, p2_ref, p3_ref, wpo_ref, bpo_ref,
                    wout_ref, bout_ref, out_ref):
    agg = (p0_ref[...] + p1_ref[...]) + (p2_ref[...] + p3_ref[...])
    h = _ssp(
        jnp.dot(agg, wpo_ref[...], preferred_element_type=jnp.float32)
        + bpo_ref[...]
    )
    out_ref[...] = (
        jnp.dot(h, wout_ref[...], preferred_element_type=jnp.float32)
        + bout_ref[...]
    )


def _out_proj4(ps, wpo, bpo, wout, bout):
    blk = pl.BlockSpec((RN, H), lambda i: (i, 0))
    return pl.pallas_call(
        _out_proj4_body,
        grid=(N // RN,),
        in_specs=[blk, blk, blk, blk,
                  pl.BlockSpec((H, D), lambda i: (0, 0)),
                  pl.BlockSpec((1, D), lambda i: (0, 0)),
                  pl.BlockSpec((D, D), lambda i: (0, 0)),
                  pl.BlockSpec((1, D), lambda i: (0, 0))],
        out_specs=pl.BlockSpec((RN, D), lambda i: (i, 0)),
        out_shape=jax.ShapeDtypeStruct((N, D), jnp.float32),
    )(*ps, wpo, bpo, wout, bout)


def kernel(node_feats, edge_feats, edge_index, Wpe1, bpe1, Wpe2, bpe2,
           Wpn, bpn, Wpo, bpo, Wout, bout):
    src = edge_index[0]
    dst = edge_index[1]
    eft = edge_feats.T
    hv = _node_proj(node_feats, Wpn, bpn.reshape(1, H))
    zinit = jnp.zeros((N, H), jnp.float32)
    b1 = bpe1.reshape(1, H)
    b2 = bpe2.reshape(1, H)
    he_a = _edge_mlp_half(eft, Wpe1, b1, Wpe2, b2, 0)
    p0a, p1a = _sc_gather_mul_scatter(hv, he_a, src, dst, zinit, 0)
    he_b = _edge_mlp_half(eft, Wpe1, b1, Wpe2, b2, 1)
    p0b, p1b = _sc_gather_mul_scatter(hv, he_b, src, dst, zinit, 1)
    return _out_proj4((p0a, p1a, p0b, p1b), Wpo, bpo.reshape(1, H),
                      Wout, bout.reshape(1, D))


# speedup vs baseline: 1.9555x; 1.0495x over previous
import functools

import jax
import jax.numpy as jnp
import numpy as np
from jax import lax
from jax.experimental import pallas as pl
from jax.experimental.pallas import tpu as pltpu
from jax.experimental.pallas import tpu_sc as plsc

N = 10000
E = 320000
D = 128
DE = 16
H = 128

NC = 2
NS = 16
NW = NC * NS
NS_SPLIT = 5
ES = E // NS_SPLIT
EPW = ES // NW
CH = 40
NIT = EPW // CH
ZR = 624
ZR_LAST = N - (NS - 1) * ZR

_LOG2 = 0.6931471805599453


def _ssp(x):
    return jnp.maximum(x, 0.0) + jnp.log1p(jnp.exp(-jnp.abs(x))) - _LOG2


def _node_proj_body(nf_ref, w_ref, b_ref, out_ref):
    out_ref[...] = (
        jnp.dot(nf_ref[...], w_ref[...], preferred_element_type=jnp.float32)
        + b_ref[...]
    )


def _edge_mlp_body(eft_ref, w1_ref, b1_ref, w2_ref, b2_ref, out_ref):
    t = lax.dot_general(eft_ref[...], w1_ref[...], (((0,), (0,)), ((), ())),
                        preferred_element_type=jnp.float32)
    t = _ssp(t + b1_ref[...])
    t = jnp.dot(t, w2_ref[...], preferred_element_type=jnp.float32)
    out_ref[...] = _ssp(t + b2_ref[...])


def _out_proj_body(p0_ref, p1_ref, wpo_ref, bpo_ref, wout_ref, bout_ref, out_ref):
    agg = p0_ref[...] + p1_ref[...]
    h = _ssp(
        jnp.dot(agg, wpo_ref[...], preferred_element_type=jnp.float32)
        + bpo_ref[...]
    )
    out_ref[...] = (
        jnp.dot(h, wout_ref[...], preferred_element_type=jnp.float32)
        + bout_ref[...]
    )


RN = 2000
REH = 1280


def _node_proj(nf, w, b):
    return pl.pallas_call(
        _node_proj_body,
        grid=(N // RN,),
        in_specs=[
            pl.BlockSpec((RN, D), lambda i: (i, 0)),
            pl.BlockSpec((D, H), lambda i: (0, 0)),
            pl.BlockSpec((1, H), lambda i: (0, 0)),
        ],
        out_specs=pl.BlockSpec((RN, H), lambda i: (i, 0)),
        out_shape=jax.ShapeDtypeStruct((N, H), jnp.float32),
    )(nf, w, b)


NBH = ES // REH


def _edge_mlp_half(eft, w1, b1, w2, b2, half):
    return pl.pallas_call(
        _edge_mlp_body,
        grid=(NBH,),
        in_specs=[
            pl.BlockSpec((DE, REH), lambda i: (0, i + half * NBH)),
            pl.BlockSpec((DE, H), lambda i: (0, 0)),
            pl.BlockSpec((1, H), lambda i: (0, 0)),
            pl.BlockSpec((H, H), lambda i: (0, 0)),
            pl.BlockSpec((1, H), lambda i: (0, 0)),
        ],
        out_specs=pl.BlockSpec((REH, H), lambda i: (i, 0)),
        out_shape=jax.ShapeDtypeStruct((ES, H), jnp.float32),
    )(eft, w1, b1, w2, b2)


def _out_proj(p0, p1, wpo, bpo, wout, bout):
    return pl.pallas_call(
        _out_proj_body,
        grid=(N // RN,),
        in_specs=[
            pl.BlockSpec((RN, H), lambda i: (i, 0)),
            pl.BlockSpec((RN, H), lambda i: (i, 0)),
            pl.BlockSpec((H, D), lambda i: (0, 0)),
            pl.BlockSpec((1, D), lambda i: (0, 0)),
            pl.BlockSpec((D, D), lambda i: (0, 0)),
            pl.BlockSpec((1, D), lambda i: (0, 0)),
        ],
        out_specs=pl.BlockSpec((RN, D), lambda i: (i, 0)),
        out_shape=jax.ShapeDtypeStruct((N, D), jnp.float32),
    )(p0, p1, wpo, bpo, wout, bout)


def _sc_gather_mul_scatter(hv, he, src3, dst3, zinit, half):
    mesh = plsc.VectorSubcoreMesh(core_axis_name="c", subcore_axis_name="s")

    @functools.partial(
        pl.kernel,
        mesh=mesh,
        out_type=[
            jax.ShapeDtypeStruct((N, H), jnp.float32),
            jax.ShapeDtypeStruct((N, H), jnp.float32),
        ],
        scratch_types=[
            pltpu.VMEM((EPW,), jnp.int32),
            pltpu.VMEM((CH,), jnp.int32),
            pltpu.VMEM((CH,), jnp.int32),
            pltpu.VMEM((CH, H), jnp.float32),
            pltpu.VMEM((CH, H), jnp.float32),
            pltpu.VMEM((CH, H), jnp.float32),
            pltpu.VMEM((CH, H), jnp.float32),
            pltpu.VMEM((CH, H), jnp.float32),
            pltpu.VMEM_SHARED((N, H), jnp.float32),
            pltpu.SemaphoreType.DMA,
            pltpu.SemaphoreType.DMA,
            pltpu.SemaphoreType.DMA,
            pltpu.SemaphoreType.DMA,
            pltpu.SemaphoreType.DMA,
        ],
    )
    def k(hv_hbm, he_hbm, src_hbm, dst_hbm, z_hbm, out0, out1,
          src_i, d0, d1, he0, he1, hvr0, hvr1, pr, agg_sh,
          g0, g1, h0, h1, s0):
        c = lax.axis_index("c")
        s = lax.axis_index("s")
        wid = s * NC + c
        row0 = s * ZR
        ebase = half * ES + wid * EPW
        hbase = wid * EPW

        def drain_f32(sem, buf):
            pltpu.make_async_copy(hv_hbm.at[pl.ds(0, CH)], buf, sem).wait()

        @pl.when(s < NS - 1)
        def _():
            pltpu.sync_copy(z_hbm.at[pl.ds(row0, ZR)], agg_sh.at[pl.ds(row0, ZR)])

        @pl.when(s == NS - 1)
        def _():
            pltpu.sync_copy(z_hbm.at[pl.ds((NS - 1) * ZR, ZR_LAST)],
                            agg_sh.at[pl.ds((NS - 1) * ZR, ZR_LAST)])

        pltpu.sync_copy(src_hbm.at[pl.ds(ebase, EPW)], src_i)
        plsc.subcore_barrier()

        pltpu.async_copy(hv_hbm.at[src_i.at[pl.ds(0, CH)]], hvr0, g0)
        pltpu.async_copy(he_hbm.at[pl.ds(hbase, CH)], he0, h0)
        pltpu.async_copy(dst_hbm.at[pl.ds(ebase, CH)], d0, h0)

        def process(i, first, hvr, he_b, d_b, gsem, hsem, n_hvr, n_he, n_d, n_g, n_h):
            @pl.when(i + 1 < NIT)
            def _():
                pltpu.async_copy(
                    hv_hbm.at[src_i.at[pl.ds((i + 1) * CH, CH)]], n_hvr, n_g)
                pltpu.async_copy(he_hbm.at[pl.ds(hbase + (i + 1) * CH, CH)],
                                 n_he, n_h)
                pltpu.async_copy(dst_hbm.at[pl.ds(ebase + (i + 1) * CH, CH)],
                                 n_d, n_h)

            drain_f32(gsem, hvr)
            drain_f32(hsem, he_b)
            pltpu.make_async_copy(dst_hbm.at[pl.ds(0, CH)], d_b, hsem).wait()

            @pl.when(jnp.logical_not(first))
            def _():
                drain_f32(s0, pr)

            def row(r, cr):
                for j in range(H // 16):
                    sl = pl.ds(j * 16, 16)
                    pr[r, sl] = hvr[r, sl] * he_b[r, sl]
                return cr

            lax.fori_loop(0, CH, row, 0)
            pltpu.async_copy(pr, agg_sh.at[d_b], s0, add=True)

        def outer(io, carry):
            i0 = io * 2
            process(i0, io == 0, hvr0, he0, d0, g0, h0, hvr1, he1, d1, g1, h1)
            process(i0 + 1, jnp.bool_(False), hvr1, he1, d1, g1, h1,
                    hvr0, he0, d0, g0, h0)
            return carry

        lax.fori_loop(0, NIT // 2, outer, 0)
        drain_f32(s0, pr)
        plsc.subcore_barrier()

        @pl.when((c == 0) & (s < NS - 1))
        def _():
            pltpu.sync_copy(agg_sh.at[pl.ds(row0, ZR)], out0.at[pl.ds(row0, ZR)])

        @pl.when((c == 0) & (s == NS - 1))
        def _():
            pltpu.sync_copy(agg_sh.at[pl.ds((NS - 1) * ZR, ZR_LAST)],
                            out0.at[pl.ds((NS - 1) * ZR, ZR_LAST)])

        @pl.when((c == 1) & (s < NS - 1))
        def _():
            pltpu.sync_copy(agg_sh.at[pl.ds(row0, ZR)], out1.at[pl.ds(row0, ZR)])

        @pl.when((c == 1) & (s == NS - 1))
        def _():
            pltpu.sync_copy(agg_sh.at[pl.ds((NS - 1) * ZR, ZR_LAST)],
                            out1.at[pl.ds((NS - 1) * ZR, ZR_LAST)])

    return k(hv, he, src3, dst3, zinit)


def _out_projN_body(*refs):
    (ps, (wpo_ref, bpo_ref, wout_ref, bout_ref, out_ref)) = (
        refs[:2 * NS_SPLIT], refs[2 * NS_SPLIT:])
    agg = ps[0][...]
    for p in ps[1:]:
        agg = agg + p[...]
    h = _ssp(
        jnp.dot(agg, wpo_ref[...], preferred_element_type=jnp.float32)
        + bpo_ref[...]
    )
    out_ref[...] = (
        jnp.dot(h, wout_ref[...], preferred_element_type=jnp.float32)
        + bout_ref[...]
    )


def _out_projN(ps, wpo, bpo, wout, bout):
    blk = pl.BlockSpec((RN, H), lambda i: (i, 0))
    return pl.pallas_call(
        _out_projN_body,
        grid=(N // RN,),
        in_specs=[blk] * (2 * NS_SPLIT) + [
            pl.BlockSpec((H, D), lambda i: (0, 0)),
            pl.BlockSpec((1, D), lambda i: (0, 0)),
            pl.BlockSpec((D, D), lambda i: (0, 0)),
            pl.BlockSpec((1, D), lambda i: (0, 0))],
        out_specs=pl.BlockSpec((RN, D), lambda i: (i, 0)),
        out_shape=jax.ShapeDtypeStruct((N, D), jnp.float32),
    )(*ps, wpo, bpo, wout, bout)


def kernel(node_feats, edge_feats, edge_index, Wpe1, bpe1, Wpe2, bpe2,
           Wpn, bpn, Wpo, bpo, Wout, bout):
    src = edge_index[0]
    dst = edge_index[1]
    eft = edge_feats.T
    hv = _node_proj(node_feats, Wpn, bpn.reshape(1, H))
    zinit = jnp.zeros((N, H), jnp.float32)
    b1 = bpe1.reshape(1, H)
    b2 = bpe2.reshape(1, H)
    parts = []
    for sp in range(NS_SPLIT):
        he_s = _edge_mlp_half(eft, Wpe1, b1, Wpe2, b2, sp)
        parts.extend(_sc_gather_mul_scatter(hv, he_s, src, dst, zinit, sp))
    return _out_projN(tuple(parts), Wpo, bpo.reshape(1, H),
                      Wout, bout.reshape(1, D))
